# baseline (speedup 1.0000x reference)
import functools

import jax
import jax.numpy as jnp
from jax.experimental import pallas as pl
from jax.experimental.pallas import tpu as pltpu

C = 2048
DEPTH = 4
ODEPTH = 2


def _add_kernel(x_hbm, pos_hbm, o_hbm, x_buf, o_buf, pos_buf,
                x_sems, o_sems, pos_sems, *, n_chunks, n_pos_chunks):
    for p in range(n_pos_chunks):
        pltpu.make_async_copy(
            pos_hbm.at[pl.ds(p * C, C)], pos_buf.at[pl.ds(p * C, C)],
            pos_sems.at[p]).start()
    for k in range(min(DEPTH, n_chunks)):
        pltpu.make_async_copy(
            x_hbm.at[pl.ds(k * C, C)], x_buf.at[k], x_sems.at[k]).start()

    def body(i, _):
        slot = jax.lax.rem(i, DEPTH)
        oslot = jax.lax.rem(i, ODEPTH)
        p = jax.lax.rem(i, n_pos_chunks)

        pltpu.make_async_copy(
            x_hbm.at[pl.ds(i * C, C)], x_buf.at[slot], x_sems.at[slot]).wait()

        @pl.when(i < n_pos_chunks)
        def _():
            pltpu.make_async_copy(
                pos_hbm.at[pl.ds(p * C, C)], pos_buf.at[pl.ds(p * C, C)],
                pos_sems.at[p]).wait()

        @pl.when(i >= ODEPTH)
        def _():
            pltpu.make_async_copy(
                o_buf.at[oslot], o_hbm.at[pl.ds((i - ODEPTH) * C, C)],
                o_sems.at[oslot]).wait()

        o_buf[oslot] = x_buf[slot] + pos_buf[pl.ds(p * C, C)]

        pltpu.make_async_copy(
            o_buf.at[oslot], o_hbm.at[pl.ds(i * C, C)], o_sems.at[oslot]
        ).start()

        @pl.when(i + DEPTH < n_chunks)
        def _():
            pltpu.make_async_copy(
                x_hbm.at[pl.ds((i + DEPTH) * C, C)], x_buf.at[slot],
                x_sems.at[slot]).start()

        return ()

    jax.lax.fori_loop(0, n_chunks, body, ())

    for j in range(max(0, n_chunks - ODEPTH), n_chunks):
        pltpu.make_async_copy(
            o_buf.at[j % ODEPTH], o_hbm.at[pl.ds(j * C, C)],
            o_sems.at[j % ODEPTH]).wait()


def kernel(x, pos_table):
    b, s, d = x.shape
    n_rows = b * s
    n_chunks = n_rows // C
    n_pos_chunks = s // C
    xf = x.reshape(n_rows, d)
    out = pl.pallas_call(
        functools.partial(_add_kernel, n_chunks=n_chunks,
                          n_pos_chunks=n_pos_chunks),
        in_specs=[
            pl.BlockSpec(memory_space=pl.ANY),
            pl.BlockSpec(memory_space=pl.ANY),
        ],
        out_specs=pl.BlockSpec(memory_space=pl.ANY),
        out_shape=jax.ShapeDtypeStruct((n_rows, d), x.dtype),
        scratch_shapes=[
            pltpu.VMEM((DEPTH, C, d), x.dtype),
            pltpu.VMEM((ODEPTH, C, d), x.dtype),
            pltpu.VMEM((s, d), x.dtype),
            pltpu.SemaphoreType.DMA((DEPTH,)),
            pltpu.SemaphoreType.DMA((ODEPTH,)),
            pltpu.SemaphoreType.DMA((s // C,)),
        ],
    )(xf, pos_table)
    return out.reshape(b, s, d)


# speedup vs baseline: 1.0049x; 1.0049x over previous
import functools

import jax
import jax.numpy as jnp
from jax.experimental import pallas as pl
from jax.experimental.pallas import tpu as pltpu

C = 2048
DEPTH = 2
ODEPTH = 4


def _add_kernel(x_hbm, pos_hbm, o_hbm, x_buf, o_buf, pos_buf,
                x_sems, o_sems, pos_sems, *, n_chunks, n_pos_chunks):
    for p in range(n_pos_chunks):
        pltpu.make_async_copy(
            pos_hbm.at[pl.ds(p * C, C)], pos_buf.at[pl.ds(p * C, C)],
            pos_sems.at[p]).start()
    for k in range(min(DEPTH, n_chunks)):
        pltpu.make_async_copy(
            x_hbm.at[pl.ds(k * C, C)], x_buf.at[k], x_sems.at[k]).start()

    def body(i, _):
        slot = jax.lax.rem(i, DEPTH)
        oslot = jax.lax.rem(i, ODEPTH)
        p = jax.lax.rem(i, n_pos_chunks)

        pltpu.make_async_copy(
            x_hbm.at[pl.ds(i * C, C)], x_buf.at[slot], x_sems.at[slot]).wait()

        @pl.when(i < n_pos_chunks)
        def _():
            pltpu.make_async_copy(
                pos_hbm.at[pl.ds(p * C, C)], pos_buf.at[pl.ds(p * C, C)],
                pos_sems.at[p]).wait()

        @pl.when(i >= ODEPTH)
        def _():
            pltpu.make_async_copy(
                o_buf.at[oslot], o_hbm.at[pl.ds((i - ODEPTH) * C, C)],
                o_sems.at[oslot]).wait()

        o_buf[oslot] = x_buf[slot] + pos_buf[pl.ds(p * C, C)]

        pltpu.make_async_copy(
            o_buf.at[oslot], o_hbm.at[pl.ds(i * C, C)], o_sems.at[oslot]
        ).start()

        @pl.when(i + DEPTH < n_chunks)
        def _():
            pltpu.make_async_copy(
                x_hbm.at[pl.ds((i + DEPTH) * C, C)], x_buf.at[slot],
                x_sems.at[slot]).start()

        return ()

    jax.lax.fori_loop(0, n_chunks, body, ())

    for j in range(max(0, n_chunks - ODEPTH), n_chunks):
        pltpu.make_async_copy(
            o_buf.at[j % ODEPTH], o_hbm.at[pl.ds(j * C, C)],
            o_sems.at[j % ODEPTH]).wait()


def kernel(x, pos_table):
    b, s, d = x.shape
    n_rows = b * s
    n_chunks = n_rows // C
    n_pos_chunks = s // C
    xf = x.reshape(n_rows, d)
    out = pl.pallas_call(
        functools.partial(_add_kernel, n_chunks=n_chunks,
                          n_pos_chunks=n_pos_chunks),
        in_specs=[
            pl.BlockSpec(memory_space=pl.ANY),
            pl.BlockSpec(memory_space=pl.ANY),
        ],
        out_specs=pl.BlockSpec(memory_space=pl.ANY),
        out_shape=jax.ShapeDtypeStruct((n_rows, d), x.dtype),
        scratch_shapes=[
            pltpu.VMEM((DEPTH, C, d), x.dtype),
            pltpu.VMEM((ODEPTH, C, d), x.dtype),
            pltpu.VMEM((s, d), x.dtype),
            pltpu.SemaphoreType.DMA((DEPTH,)),
            pltpu.SemaphoreType.DMA((ODEPTH,)),
            pltpu.SemaphoreType.DMA((s // C,)),
        ],
    )(xf, pos_table)
    return out.reshape(b, s, d)


# speedup vs baseline: 1.0524x; 1.0473x over previous
import functools

import jax
import jax.numpy as jnp
from jax.experimental import pallas as pl
from jax.experimental.pallas import tpu as pltpu

C = 2048
DEPTH = 3
ODEPTH = 3


def _add_kernel(x_hbm, pos_hbm, o_hbm, x_buf, o_buf, pos_buf,
                x_sems, o_sems, pos_sems, *, n_chunks, n_pos_chunks):
    for p in range(n_pos_chunks):
        pltpu.make_async_copy(
            pos_hbm.at[pl.ds(p * C, C)], pos_buf.at[pl.ds(p * C, C)],
            pos_sems.at[p]).start()
    for k in range(min(DEPTH, n_chunks)):
        pltpu.make_async_copy(
            x_hbm.at[pl.ds(k * C, C)], x_buf.at[k], x_sems.at[k]).start()

    def body(i, _):
        slot = jax.lax.rem(i, DEPTH)
        oslot = jax.lax.rem(i, ODEPTH)
        p = jax.lax.rem(i, n_pos_chunks)

        pltpu.make_async_copy(
            x_hbm.at[pl.ds(i * C, C)], x_buf.at[slot], x_sems.at[slot]).wait()

        @pl.when(i < n_pos_chunks)
        def _():
            pltpu.make_async_copy(
                pos_hbm.at[pl.ds(p * C, C)], pos_buf.at[pl.ds(p * C, C)],
                pos_sems.at[p]).wait()

        @pl.when(i >= ODEPTH)
        def _():
            pltpu.make_async_copy(
                o_buf.at[oslot], o_hbm.at[pl.ds((i - ODEPTH) * C, C)],
                o_sems.at[oslot]).wait()

        o_buf[oslot] = x_buf[slot] + pos_buf[pl.ds(p * C, C)]

        pltpu.make_async_copy(
            o_buf.at[oslot], o_hbm.at[pl.ds(i * C, C)], o_sems.at[oslot]
        ).start()

        @pl.when(i + DEPTH < n_chunks)
        def _():
            pltpu.make_async_copy(
                x_hbm.at[pl.ds((i + DEPTH) * C, C)], x_buf.at[slot],
                x_sems.at[slot]).start()

        return ()

    jax.lax.fori_loop(0, n_chunks, body, ())

    for j in range(max(0, n_chunks - ODEPTH), n_chunks):
        pltpu.make_async_copy(
            o_buf.at[j % ODEPTH], o_hbm.at[pl.ds(j * C, C)],
            o_sems.at[j % ODEPTH]).wait()


def kernel(x, pos_table):
    b, s, d = x.shape
    n_rows = b * s
    n_chunks = n_rows // C
    n_pos_chunks = s // C
    xf = x.reshape(n_rows, d)
    out = pl.pallas_call(
        functools.partial(_add_kernel, n_chunks=n_chunks,
                          n_pos_chunks=n_pos_chunks),
        in_specs=[
            pl.BlockSpec(memory_space=pl.ANY),
            pl.BlockSpec(memory_space=pl.ANY),
        ],
        out_specs=pl.BlockSpec(memory_space=pl.ANY),
        out_shape=jax.ShapeDtypeStruct((n_rows, d), x.dtype),
        scratch_shapes=[
            pltpu.VMEM((DEPTH, C, d), x.dtype),
            pltpu.VMEM((ODEPTH, C, d), x.dtype),
            pltpu.VMEM((s, d), x.dtype),
            pltpu.SemaphoreType.DMA((DEPTH,)),
            pltpu.SemaphoreType.DMA((ODEPTH,)),
            pltpu.SemaphoreType.DMA((s // C,)),
        ],
    )(xf, pos_table)
    return out.reshape(b, s, d)
